# baseline (speedup 1.0000x reference)
import functools

import jax
import jax.numpy as jnp
from jax import lax
from jax.experimental import pallas as pl
from jax.experimental.pallas import tpu as pltpu
from jax.experimental.pallas import tpu_sc as plsc

N = 10000
E = 320000
D = 128

NC = 2
NS = 16
NW = NC * NS
L = 16

N_PAD = 10240
CHUNK = 128
EPT = 10240
E_PAD = EPT * NW
NBUF = 2
IDXB = 32
TOTB = E_PAD // CHUNK
BPP = 2 * EPT // CHUNK
AGG_NC = 1
B0 = BPP
B1 = BPP - B0

_mesh = plsc.VectorSubcoreMesh(
    core_axis_name="c", subcore_axis_name="s", num_cores=NC, num_subcores=NS)


@functools.partial(
    pl.kernel,
    out_type=jax.ShapeDtypeStruct((NW, N_PAD), jnp.float32),
    mesh=_mesh,
    scratch_types=[
        pltpu.VMEM((EPT,), jnp.int32),
        pltpu.VMEM((N_PAD,), jnp.float32),
    ],
    compiler_params=pltpu.CompilerParams(needs_layout_passes=False),
)
def _deg_sc(dst_hbm, zeros_hbm, out_hbm, dst_v, tab_v):
    c = lax.axis_index("c")
    s = lax.axis_index("s")
    wid = c * NS + s

    pltpu.sync_copy(zeros_hbm, tab_v)
    pltpu.sync_copy(dst_hbm.at[pl.ds(wid * EPT, EPT)], dst_v)

    ones = jnp.full((L,), 1.0, jnp.float32)

    def body(i, _):
        d = dst_v[pl.ds(i * L, L)]
        plsc.addupdate_scatter(tab_v, [d], ones)
        return _

    lax.fori_loop(0, EPT // L, body, None)

    pltpu.sync_copy(tab_v, out_hbm.at[wid])


_agg_mesh = plsc.VectorSubcoreMesh(
    core_axis_name="c", subcore_axis_name="s", num_cores=AGG_NC,
    num_subcores=NS)


@functools.partial(
    pl.kernel,
    out_type=jax.ShapeDtypeStruct((AGG_NC, N_PAD, D), jnp.float32),
    mesh=_agg_mesh,
    scratch_types=[
        pltpu.VMEM((IDXB, CHUNK), jnp.int32),
        pltpu.VMEM((IDXB, CHUNK), jnp.int32),
        pltpu.VMEM((CHUNK, D), jnp.float32),
        pltpu.VMEM((CHUNK, D), jnp.float32),
        pltpu.VMEM_SHARED((N_PAD, D), jnp.float32),
        pltpu.SemaphoreType.DMA,
    ],
    compiler_params=pltpu.CompilerParams(needs_layout_passes=False),
)
def _agg_sc(y_hbm, src_hbm, dst_hbm, out_hbm, src_v, dst_v,
            rb0, rb1, tab_sh, sem):
    ring = (rb0, rb1)
    c = lax.axis_index("c")
    s = lax.axis_index("s")
    rows = N_PAD // NS
    stages = jnp.where(c == 0, B0 // IDXB, B1 // IDXB)
    tile_base = jnp.where(c == 0, s * B0, NS * B0 + s * B1)

    def _guarded(fn):
        if B1 > 0:
            fn()
        else:
            pl.when(c == 0)(fn)

    zero16 = jnp.zeros((L,), jnp.float32)

    def zbody(i, _):
        r = i // (D // L)
        k = i - r * (D // L)
        rb0[r, pl.ds(k * L, L)] = zero16
        return _

    def _zero_tab():
        lax.fori_loop(0, CHUNK * (D // L), zbody, None)
        for t in range(rows // CHUNK):
            pltpu.sync_copy(rb0, tab_sh.at[pl.ds(s * rows + t * CHUNK, CHUNK)])

    _guarded(_zero_tab)
    plsc.subcore_barrier()

    def stage(h, _):
        off = tile_base + h * IDXB
        pltpu.sync_copy(src_hbm.at[pl.ds(off, IDXB)], src_v)
        pltpu.sync_copy(dst_hbm.at[pl.ds(off, IDXB)], dst_v)
        for b in range(NBUF):
            pltpu.async_copy(y_hbm.at[src_v.at[b]], ring[b], sem)

        def outer(j, _):
            for b in range(NBUF):
                g = j * NBUF + b
                pltpu.make_async_copy(y_hbm.at[src_v.at[g]], ring[b],
                                      sem).wait()
                pltpu.sync_copy(ring[b], tab_sh.at[dst_v.at[g]], add=True)

                @pl.when(g + NBUF < IDXB)
                def _():
                    pltpu.async_copy(y_hbm.at[src_v.at[g + NBUF]], ring[b],
                                     sem)
            return _

        lax.fori_loop(0, IDXB // NBUF, outer, None)
        return _

    lax.fori_loop(0, stages, stage, None)

    plsc.subcore_barrier()
    _guarded(lambda: pltpu.sync_copy(
        tab_sh.at[pl.ds(s * rows, rows)],
        out_hbm.at[c].at[pl.ds(s * rows, rows)]))


_RB = 1024


def _tca_body(prev_ref, raw_ref, w_ref, u_ref):
    u_ref[...] = jnp.dot(prev_ref[...], w_ref[...],
                         preferred_element_type=jnp.float32) + raw_ref[...]


def _tc_a(prev_p, raw_p, W):
    return pl.pallas_call(
        _tca_body,
        grid=(N_PAD // _RB,),
        in_specs=[
            pl.BlockSpec((_RB, D), lambda i: (i, 0)),
            pl.BlockSpec((_RB, D), lambda i: (i, 0)),
            pl.BlockSpec((D, D), lambda i: (0, 0)),
        ],
        out_specs=pl.BlockSpec((_RB, D), lambda i: (i, 0)),
        out_shape=jax.ShapeDtypeStruct((N_PAD, D), jnp.float32),
    )(prev_p, raw_p, W)


def _norm_from(deg_ref):
    d = jnp.sum(deg_ref[...], axis=1, keepdims=True)
    return lax.rsqrt(jnp.maximum(d, 1.0))


def _tcb_body(u_ref, deg_ref, y_ref):
    y_ref[...] = u_ref[...] * _norm_from(deg_ref)


def _tc_b(U, deg_t):
    return pl.pallas_call(
        _tcb_body,
        grid=(N_PAD // _RB,),
        in_specs=[
            pl.BlockSpec((_RB, D), lambda i: (i, 0)),
            pl.BlockSpec((_RB, NW), lambda i: (i, 0)),
        ],
        out_specs=pl.BlockSpec((_RB, D), lambda i: (i, 0)),
        out_shape=jax.ShapeDtypeStruct((N_PAD, D), jnp.float32),
    )(U, deg_t)


_NPC = NC if B1 > 0 else 1


def _tcc_body(part_ref, deg_ref, out_ref):
    acc = part_ref[0]
    for i in range(1, _NPC):
        acc = acc + part_ref[i]
    out_ref[...] = jnp.maximum(acc * _norm_from(deg_ref), 0.0)


_RC = 1000


def _tc_c(part, deg_t):
    return pl.pallas_call(
        _tcc_body,
        grid=(N // _RC,),
        in_specs=[
            pl.BlockSpec((_NPC, _RC, D), lambda i: (0, i, 0)),
            pl.BlockSpec((_RC, NW), lambda i: (i, 0)),
        ],
        out_specs=pl.BlockSpec((_RC, D), lambda i: (i, 0)),
        out_shape=jax.ShapeDtypeStruct((N, D), jnp.float32),
    )(part, deg_t)


def kernel(prev, raw, edge_index, W):
    src = edge_index[0]
    dst = edge_index[1]
    src_p = jnp.pad(src, (0, E_PAD - E))
    pad_dst = N + (jnp.arange(E_PAD - E, dtype=jnp.int32) % (N_PAD - N))
    dst_p = jnp.concatenate([dst, pad_dst])
    prev_p = jnp.pad(prev, ((0, N_PAD - N), (0, 0)))
    raw_p = jnp.pad(raw, ((0, N_PAD - N), (0, 0)))

    zeros1 = jnp.zeros((N_PAD,), jnp.float32)

    deg2 = _deg_sc(dst_p, zeros1)
    U = _tc_a(prev_p, raw_p, W)
    deg_t = deg2.T
    Y = _tc_b(U, deg_t)
    src3 = src_p.reshape(TOTB, CHUNK)
    dst3 = dst_p.reshape(TOTB, CHUNK)
    part = _agg_sc(Y, src3, dst3)
    return _tc_c(part, deg_t)


# speedup vs baseline: 1.5105x; 1.5105x over previous
import functools

import jax
import jax.numpy as jnp
from jax import lax
from jax.experimental import pallas as pl
from jax.experimental.pallas import tpu as pltpu
from jax.experimental.pallas import tpu_sc as plsc

N = 10000
E = 320000
D = 128

NC = 2
NS = 16
NW = NC * NS
L = 16

N_PAD = 10240
CHUNK = 128
EPT = 10240
E_PAD = EPT * NW
NBUF = 2
IDXB = 16
TOTB = E_PAD // CHUNK
BPP = 2 * EPT // CHUNK
AGG_NC = 2
B0 = 144
B1 = BPP - B0

_mesh = plsc.VectorSubcoreMesh(
    core_axis_name="c", subcore_axis_name="s", num_cores=NC, num_subcores=NS)


@functools.partial(
    pl.kernel,
    out_type=jax.ShapeDtypeStruct((NW, N_PAD), jnp.float32),
    mesh=_mesh,
    scratch_types=[
        pltpu.VMEM((EPT,), jnp.int32),
        pltpu.VMEM((N_PAD,), jnp.float32),
    ],
    compiler_params=pltpu.CompilerParams(needs_layout_passes=False),
)
def _deg_sc(dst_hbm, zeros_hbm, out_hbm, dst_v, tab_v):
    c = lax.axis_index("c")
    s = lax.axis_index("s")
    wid = c * NS + s

    pltpu.sync_copy(zeros_hbm, tab_v)
    pltpu.sync_copy(dst_hbm.at[pl.ds(wid * EPT, EPT)], dst_v)

    ones = jnp.full((L,), 1.0, jnp.float32)

    def body(i, _):
        d = dst_v[pl.ds(i * L, L)]
        plsc.addupdate_scatter(tab_v, [d], ones)
        return _

    lax.fori_loop(0, EPT // L, body, None)

    pltpu.sync_copy(tab_v, out_hbm.at[wid])


_agg_mesh = plsc.VectorSubcoreMesh(
    core_axis_name="c", subcore_axis_name="s", num_cores=AGG_NC,
    num_subcores=NS)


@functools.partial(
    pl.kernel,
    out_type=jax.ShapeDtypeStruct((AGG_NC, N_PAD, D), jnp.float32),
    mesh=_agg_mesh,
    scratch_types=[
        pltpu.VMEM((IDXB, CHUNK), jnp.int32),
        pltpu.VMEM((IDXB, CHUNK), jnp.int32),
        pltpu.VMEM((CHUNK, D), jnp.float32),
        pltpu.VMEM((CHUNK, D), jnp.float32),
        pltpu.VMEM_SHARED((N_PAD, D), jnp.float32),
        pltpu.SemaphoreType.DMA,
    ],
    compiler_params=pltpu.CompilerParams(needs_layout_passes=False),
)
def _agg_sc(y_hbm, src_hbm, dst_hbm, out_hbm, src_v, dst_v,
            rb0, rb1, tab_sh, sem):
    ring = (rb0, rb1)
    c = lax.axis_index("c")
    s = lax.axis_index("s")
    rows = N_PAD // NS
    stages = jnp.where(c == 0, B0 // IDXB, B1 // IDXB)
    tile_base = jnp.where(c == 0, s * B0, NS * B0 + s * B1)

    def _guarded(fn):
        if B1 > 0:
            fn()
        else:
            pl.when(c == 0)(fn)

    zero16 = jnp.zeros((L,), jnp.float32)

    def zbody(i, _):
        r = i // (D // L)
        k = i - r * (D // L)
        rb0[r, pl.ds(k * L, L)] = zero16
        return _

    def _zero_tab():
        lax.fori_loop(0, CHUNK * (D // L), zbody, None)
        for t in range(rows // CHUNK):
            pltpu.sync_copy(rb0, tab_sh.at[pl.ds(s * rows + t * CHUNK, CHUNK)])

    _guarded(_zero_tab)
    plsc.subcore_barrier()

    def stage(h, _):
        off = tile_base + h * IDXB
        pltpu.sync_copy(src_hbm.at[pl.ds(off, IDXB)], src_v)
        pltpu.sync_copy(dst_hbm.at[pl.ds(off, IDXB)], dst_v)
        for b in range(NBUF):
            pltpu.async_copy(y_hbm.at[src_v.at[b]], ring[b], sem)

        def outer(j, _):
            for b in range(NBUF):
                g = j * NBUF + b
                pltpu.make_async_copy(y_hbm.at[src_v.at[g]], ring[b],
                                      sem).wait()
                pltpu.sync_copy(ring[b], tab_sh.at[dst_v.at[g]], add=True)

                @pl.when(g + NBUF < IDXB)
                def _():
                    pltpu.async_copy(y_hbm.at[src_v.at[g + NBUF]], ring[b],
                                     sem)
            return _

        lax.fori_loop(0, IDXB // NBUF, outer, None)
        return _

    lax.fori_loop(0, stages, stage, None)

    plsc.subcore_barrier()
    _guarded(lambda: pltpu.sync_copy(
        tab_sh.at[pl.ds(s * rows, rows)],
        out_hbm.at[c].at[pl.ds(s * rows, rows)]))


_RB = 1024


def _tca_body(prev_ref, raw_ref, w_ref, u_ref):
    u_ref[...] = jnp.dot(prev_ref[...], w_ref[...],
                         preferred_element_type=jnp.float32) + raw_ref[...]


def _tc_a(prev_p, raw_p, W):
    return pl.pallas_call(
        _tca_body,
        grid=(N_PAD // _RB,),
        in_specs=[
            pl.BlockSpec((_RB, D), lambda i: (i, 0)),
            pl.BlockSpec((_RB, D), lambda i: (i, 0)),
            pl.BlockSpec((D, D), lambda i: (0, 0)),
        ],
        out_specs=pl.BlockSpec((_RB, D), lambda i: (i, 0)),
        out_shape=jax.ShapeDtypeStruct((N_PAD, D), jnp.float32),
    )(prev_p, raw_p, W)


def _norm_from(deg_ref):
    d = jnp.sum(deg_ref[...], axis=1, keepdims=True)
    return lax.rsqrt(jnp.maximum(d, 1.0))


def _tcb_body(u_ref, deg_ref, y_ref):
    y_ref[...] = u_ref[...] * _norm_from(deg_ref)


def _tc_b(U, deg_t):
    return pl.pallas_call(
        _tcb_body,
        grid=(N_PAD // _RB,),
        in_specs=[
            pl.BlockSpec((_RB, D), lambda i: (i, 0)),
            pl.BlockSpec((_RB, NW), lambda i: (i, 0)),
        ],
        out_specs=pl.BlockSpec((_RB, D), lambda i: (i, 0)),
        out_shape=jax.ShapeDtypeStruct((N_PAD, D), jnp.float32),
    )(U, deg_t)


_NPC = NC if B1 > 0 else 1


def _tcc_body(part_ref, deg_ref, out_ref):
    acc = part_ref[0]
    for i in range(1, _NPC):
        acc = acc + part_ref[i]
    out_ref[...] = jnp.maximum(acc * _norm_from(deg_ref), 0.0)


_RC = 1000


def _tc_c(part, deg_t):
    return pl.pallas_call(
        _tcc_body,
        grid=(N // _RC,),
        in_specs=[
            pl.BlockSpec((_NPC, _RC, D), lambda i: (0, i, 0)),
            pl.BlockSpec((_RC, NW), lambda i: (i, 0)),
        ],
        out_specs=pl.BlockSpec((_RC, D), lambda i: (i, 0)),
        out_shape=jax.ShapeDtypeStruct((N, D), jnp.float32),
    )(part, deg_t)


def kernel(prev, raw, edge_index, W):
    src = edge_index[0]
    dst = edge_index[1]
    src_p = jnp.pad(src, (0, E_PAD - E))
    pad_dst = N + (jnp.arange(E_PAD - E, dtype=jnp.int32) % (N_PAD - N))
    dst_p = jnp.concatenate([dst, pad_dst])
    prev_p = jnp.pad(prev, ((0, N_PAD - N), (0, 0)))
    raw_p = jnp.pad(raw, ((0, N_PAD - N), (0, 0)))

    zeros1 = jnp.zeros((N_PAD,), jnp.float32)

    deg2 = _deg_sc(dst_p, zeros1)
    U = _tc_a(prev_p, raw_p, W)
    deg_t = deg2.T
    Y = _tc_b(U, deg_t)
    src3 = src_p.reshape(TOTB, CHUNK)
    dst3 = dst_p.reshape(TOTB, CHUNK)
    part = _agg_sc(Y, src3, dst3)
    return _tc_c(part, deg_t)


# speedup vs baseline: 1.5392x; 1.0190x over previous
import functools

import jax
import jax.numpy as jnp
from jax import lax
from jax.experimental import pallas as pl
from jax.experimental.pallas import tpu as pltpu
from jax.experimental.pallas import tpu_sc as plsc

N = 10000
E = 320000
D = 128

NC = 2
NS = 16
NW = NC * NS
L = 16

N_PAD = 10240
CHUNK = 128
EPT = 10240
E_PAD = EPT * NW
NBUF = 2
IDXB = 8
TOTB = E_PAD // CHUNK
BPP = 2 * EPT // CHUNK
AGG_NC = 2
B0 = 152
B1 = BPP - B0

_mesh = plsc.VectorSubcoreMesh(
    core_axis_name="c", subcore_axis_name="s", num_cores=NC, num_subcores=NS)


@functools.partial(
    pl.kernel,
    out_type=jax.ShapeDtypeStruct((NW, N_PAD), jnp.float32),
    mesh=_mesh,
    scratch_types=[
        pltpu.VMEM((EPT,), jnp.int32),
        pltpu.VMEM((N_PAD,), jnp.float32),
    ],
    compiler_params=pltpu.CompilerParams(needs_layout_passes=False),
)
def _deg_sc(dst_hbm, zeros_hbm, out_hbm, dst_v, tab_v):
    c = lax.axis_index("c")
    s = lax.axis_index("s")
    wid = c * NS + s

    pltpu.sync_copy(zeros_hbm, tab_v)
    pltpu.sync_copy(dst_hbm.at[pl.ds(wid * EPT, EPT)], dst_v)

    ones = jnp.full((L,), 1.0, jnp.float32)

    def body(i, _):
        d = dst_v[pl.ds(i * L, L)]
        plsc.addupdate_scatter(tab_v, [d], ones)
        return _

    lax.fori_loop(0, EPT // L, body, None)

    pltpu.sync_copy(tab_v, out_hbm.at[wid])


_agg_mesh = plsc.VectorSubcoreMesh(
    core_axis_name="c", subcore_axis_name="s", num_cores=AGG_NC,
    num_subcores=NS)


@functools.partial(
    pl.kernel,
    out_type=jax.ShapeDtypeStruct((AGG_NC, N_PAD, D), jnp.float32),
    mesh=_agg_mesh,
    scratch_types=[
        pltpu.VMEM((IDXB, CHUNK), jnp.int32),
        pltpu.VMEM((IDXB, CHUNK), jnp.int32),
        pltpu.VMEM((CHUNK, D), jnp.float32),
        pltpu.VMEM((CHUNK, D), jnp.float32),
        pltpu.VMEM_SHARED((N_PAD, D), jnp.float32),
        pltpu.SemaphoreType.DMA,
    ],
    compiler_params=pltpu.CompilerParams(needs_layout_passes=False),
)
def _agg_sc(y_hbm, src_hbm, dst_hbm, out_hbm, src_v, dst_v,
            rb0, rb1, tab_sh, sem):
    ring = (rb0, rb1)
    c = lax.axis_index("c")
    s = lax.axis_index("s")
    rows = N_PAD // NS
    stages = jnp.where(c == 0, B0 // IDXB, B1 // IDXB)
    tile_base = jnp.where(c == 0, s * B0, NS * B0 + s * B1)

    def _guarded(fn):
        if B1 > 0:
            fn()
        else:
            pl.when(c == 0)(fn)

    zero16 = jnp.zeros((L,), jnp.float32)

    def zbody(i, _):
        r = i // (D // L)
        k = i - r * (D // L)
        rb0[r, pl.ds(k * L, L)] = zero16
        return _

    def _zero_tab():
        lax.fori_loop(0, CHUNK * (D // L), zbody, None)
        for t in range(rows // CHUNK):
            pltpu.sync_copy(rb0, tab_sh.at[pl.ds(s * rows + t * CHUNK, CHUNK)])

    _guarded(_zero_tab)
    plsc.subcore_barrier()

    def stage(h, _):
        off = tile_base + h * IDXB
        pltpu.sync_copy(src_hbm.at[pl.ds(off, IDXB)], src_v)
        pltpu.sync_copy(dst_hbm.at[pl.ds(off, IDXB)], dst_v)
        for b in range(NBUF):
            pltpu.async_copy(y_hbm.at[src_v.at[b]], ring[b], sem)

        def outer(j, _):
            for b in range(NBUF):
                g = j * NBUF + b
                pltpu.make_async_copy(y_hbm.at[src_v.at[g]], ring[b],
                                      sem).wait()
                pltpu.sync_copy(ring[b], tab_sh.at[dst_v.at[g]], add=True)

                @pl.when(g + NBUF < IDXB)
                def _():
                    pltpu.async_copy(y_hbm.at[src_v.at[g + NBUF]], ring[b],
                                     sem)
            return _

        lax.fori_loop(0, IDXB // NBUF, outer, None)
        return _

    lax.fori_loop(0, stages, stage, None)

    plsc.subcore_barrier()
    _guarded(lambda: pltpu.sync_copy(
        tab_sh.at[pl.ds(s * rows, rows)],
        out_hbm.at[c].at[pl.ds(s * rows, rows)]))


_RB = 1024


def _tca_body(prev_ref, raw_ref, w_ref, u_ref):
    u_ref[...] = jnp.dot(prev_ref[...], w_ref[...],
                         preferred_element_type=jnp.float32) + raw_ref[...]


def _tc_a(prev_p, raw_p, W):
    return pl.pallas_call(
        _tca_body,
        grid=(N_PAD // _RB,),
        in_specs=[
            pl.BlockSpec((_RB, D), lambda i: (i, 0)),
            pl.BlockSpec((_RB, D), lambda i: (i, 0)),
            pl.BlockSpec((D, D), lambda i: (0, 0)),
        ],
        out_specs=pl.BlockSpec((_RB, D), lambda i: (i, 0)),
        out_shape=jax.ShapeDtypeStruct((N_PAD, D), jnp.float32),
    )(prev_p, raw_p, W)


def _norm_from(deg_ref):
    d = jnp.sum(deg_ref[...], axis=1, keepdims=True)
    return lax.rsqrt(jnp.maximum(d, 1.0))


def _tcb_body(u_ref, deg_ref, y_ref):
    y_ref[...] = u_ref[...] * _norm_from(deg_ref)


def _tc_b(U, deg_t):
    return pl.pallas_call(
        _tcb_body,
        grid=(N_PAD // _RB,),
        in_specs=[
            pl.BlockSpec((_RB, D), lambda i: (i, 0)),
            pl.BlockSpec((_RB, NW), lambda i: (i, 0)),
        ],
        out_specs=pl.BlockSpec((_RB, D), lambda i: (i, 0)),
        out_shape=jax.ShapeDtypeStruct((N_PAD, D), jnp.float32),
    )(U, deg_t)


_NPC = NC if B1 > 0 else 1


def _tcc_body(part_ref, deg_ref, out_ref):
    acc = part_ref[0]
    for i in range(1, _NPC):
        acc = acc + part_ref[i]
    out_ref[...] = jnp.maximum(acc * _norm_from(deg_ref), 0.0)


_RC = 1000


def _tc_c(part, deg_t):
    return pl.pallas_call(
        _tcc_body,
        grid=(N // _RC,),
        in_specs=[
            pl.BlockSpec((_NPC, _RC, D), lambda i: (0, i, 0)),
            pl.BlockSpec((_RC, NW), lambda i: (i, 0)),
        ],
        out_specs=pl.BlockSpec((_RC, D), lambda i: (i, 0)),
        out_shape=jax.ShapeDtypeStruct((N, D), jnp.float32),
    )(part, deg_t)


def kernel(prev, raw, edge_index, W):
    src = edge_index[0]
    dst = edge_index[1]
    src_p = jnp.pad(src, (0, E_PAD - E))
    pad_dst = N + (jnp.arange(E_PAD - E, dtype=jnp.int32) % (N_PAD - N))
    dst_p = jnp.concatenate([dst, pad_dst])
    prev_p = jnp.pad(prev, ((0, N_PAD - N), (0, 0)))
    raw_p = jnp.pad(raw, ((0, N_PAD - N), (0, 0)))

    zeros1 = jnp.zeros((N_PAD,), jnp.float32)

    deg2 = _deg_sc(dst_p, zeros1)
    U = _tc_a(prev_p, raw_p, W)
    deg_t = deg2.T
    Y = _tc_b(U, deg_t)
    src3 = src_p.reshape(TOTB, CHUNK)
    dst3 = dst_p.reshape(TOTB, CHUNK)
    part = _agg_sc(Y, src3, dst3)
    return _tc_c(part, deg_t)
